# baseline (speedup 1.0000x reference)
import dataclasses

import jax
import jax.numpy as jnp
from jax.experimental import pallas as pl
from jax.experimental.pallas import tpu as pltpu
from jax.experimental.pallas import tpu_sc as plsc

_SC_PARAMS = pltpu.CompilerParams()
if "needs_layout_passes" in pltpu.CompilerParams.__dataclass_fields__:
    _SC_PARAMS = dataclasses.replace(_SC_PARAMS, needs_layout_passes=False)

_BATCH = 2
_W = 128


def _vq_tc_body(x_ref, cb_ref, idx_ref, loss_ref):
    i = pl.program_id(0)
    xc = jnp.concatenate([x_ref[b] for b in range(_BATCH)], axis=1)
    cb = cb_ref[...]
    n = xc.shape[1]
    mm = jax.lax.dot_general(
        cb, xc, (((0,), (0,)), ((), ())),
        preferred_element_type=jnp.float32)
    xsq = jnp.sum(xc * xc, axis=0, keepdims=True)
    cbsq = jnp.sum(cb * cb, axis=0).reshape(-1, 1)
    s_h = 128
    n_s = mm.shape[0] // s_h

    def slab(s):
        return (xsq - 2.0 * mm[s * s_h:(s + 1) * s_h]) + cbsq[s * s_h:
                                                              (s + 1) * s_h]

    best = slab(0)
    best_s = jnp.zeros(best.shape, jnp.int32)
    for s in range(1, n_s):
        d_s = slab(s)
        upd = d_s < best
        best = jnp.where(upd, d_s, best)
        best_s = jnp.where(upd, jnp.int32(s), best_s)
    m = jnp.min(best, axis=0, keepdims=True)
    sub = jax.lax.broadcasted_iota(jnp.int32, best.shape, 0)
    cand = best_s * s_h + sub
    masked = jnp.where(best == m, cand, jnp.int32(2 ** 30))
    idx_ref[0, 0, :] = jnp.min(masked, axis=0)
    part = jnp.sum(m).reshape(1, 1)

    @pl.when(i == 0)
    def _init():
        loss_ref[...] = part

    @pl.when(i != 0)
    def _acc():
        loss_ref[...] += part


def _argmin_loss(xt, cbt, chunk=0, n_chunks=1, *, interpret=False):
    b, h, m = xt.shape
    grid = b // _BATCH // n_chunks
    base = chunk * grid
    return pl.pallas_call(
        _vq_tc_body,
        grid=(grid,),
        in_specs=[
            pl.BlockSpec((_BATCH, h, m), lambda i: (base + i, 0, 0)),
            pl.BlockSpec(cbt.shape, lambda i: (0, 0)),
        ],
        out_specs=[
            pl.BlockSpec((1, 1, _BATCH * m), lambda i: (i, 0, 0)),
            pl.BlockSpec((1, 1), lambda i: (0, 0)),
        ],
        out_shape=[
            jax.ShapeDtypeStruct((grid, 1, _BATCH * m), jnp.int32),
            jax.ShapeDtypeStruct((1, 1), jnp.float32),
        ],
        interpret=interpret,
    )(xt, cbt)


def _gather_rows_t(cbt, idx2, b, m, b0=0, carry=None):
    h = cbt.shape[0]
    n_bp = idx2.shape[0]
    w = idx2.shape[2]
    b_out = b0 + 2 * n_bp
    mesh = plsc.VectorSubcoreMesh(core_axis_name="core",
                                  subcore_axis_name="subcore")
    n_hg = h // 8
    bp_per_u = n_bp // 4

    args = (cbt, idx2) if carry is None else (cbt, idx2, carry)

    @pl.kernel(out_type=jax.ShapeDtypeStruct((b_out, h, m), cbt.dtype),
               mesh=mesh,
               compiler_params=_SC_PARAMS,
               scratch_types=[pltpu.VMEM((8, cbt.shape[1]), cbt.dtype),
                              pltpu.VMEM((1, 1, w), jnp.int32),
                              pltpu.VMEM((1, 8, m), cbt.dtype),
                              pltpu.VMEM((1, 8, m), cbt.dtype),
                              pltpu.SemaphoreType.DMA,
                              pltpu.SemaphoreType.DMA,
                              pltpu.SemaphoreType.DMA,
                              pltpu.SemaphoreType.DMA,
                              pltpu.SemaphoreType.DMA])
    def _sc_gather(cb_hbm, i_hbm, *rest):
        (o_hbm, cb_v, idx_v, ob0, ob1, sc, si, sw0, sw1, scr) = rest[-10:]
        c_hbm = rest[0] if carry is not None else None
        u = (jax.lax.axis_index("core") * 16
             + jax.lax.axis_index("subcore"))
        hg = u % n_hg
        bpg = u // n_hg
        obufs = (ob0, ob1)
        sw = (sw0, sw1)
        carry_copy = None
        if carry is not None:
            carry_copy = pltpu.make_async_copy(
                c_hbm.at[pl.ds(u // 2, 1), pl.ds(32 * (u % 2), 32), :],
                o_hbm.at[pl.ds(u // 2, 1), pl.ds(32 * (u % 2), 32), :],
                scr)
            carry_copy.start()
        pltpu.async_copy(cb_hbm.at[pl.ds(hg * 8, 8), :], cb_v, sc).wait()

        def out_copy(bp, bi):
            return pltpu.make_async_copy(
                obufs[bi],
                o_hbm.at[pl.ds(b0 + 2 * bp + bi, 1), pl.ds(hg * 8, 8), :],
                sw[bi])

        @pl.loop(0, bp_per_u)
        def _chunk(t):
            bp = bpg * bp_per_u + t
            pltpu.async_copy(i_hbm.at[pl.ds(bp, 1), :, :], idx_v, si).wait()
            for bi in range(2):
                @pl.when(t >= 1)
                def _wait_prev(bi=bi):
                    out_copy(bp - 1, bi).wait()
                for hh in range(8):
                    hh_vec = jnp.full((16,), hh, jnp.int32)
                    for j in range(m // 16):
                        g = plsc.load_gather(
                            cb_v,
                            [hh_vec, idx_v[0, 0, pl.ds(bi * m + j * 16, 16)]])
                        obufs[bi][0, hh, pl.ds(j * 16, 16)] = g
                out_copy(bp, bi).start()

        for bi in range(2):
            out_copy(bpg * bp_per_u + bp_per_u - 1, bi).wait()
        if carry is not None:
            carry_copy.wait()

    return _sc_gather(*args)


def _gather_rows(codebook, idx, h_out):
    n = idx.shape[0]
    k = codebook.shape[0]
    h_pad = 128
    codebook = jnp.concatenate(
        [codebook, jnp.zeros((k, h_pad - h_out), codebook.dtype)], axis=1)
    idx2 = idx.reshape(1, n)
    mesh = plsc.VectorSubcoreMesh(core_axis_name="core",
                                  subcore_axis_name="subcore")
    n_blocks = n // _W
    n_units = 32
    max_per_unit = -(-n_blocks // n_units)

    @pl.kernel(out_type=jax.ShapeDtypeStruct((n, h_pad), codebook.dtype),
               mesh=mesh,
               scratch_types=[pltpu.VMEM((2, _W), jnp.int32),
                              pltpu.VMEM((2, _W, h_pad), codebook.dtype),
                              pltpu.SemaphoreType.DMA,
                              pltpu.SemaphoreType.DMA,
                              pltpu.SemaphoreType.DMA,
                              pltpu.SemaphoreType.DMA,
                              pltpu.SemaphoreType.DMA])
    def _sc_gather(cb_hbm, i_hbm, o_hbm, idx_v, scr, si0, si1, sg, sw0, sw1):
        u = (jax.lax.axis_index("core") * 16
             + jax.lax.axis_index("subcore"))
        si = (si0, si1)
        sw = (sw0, sw1)

        def blk(s):
            return s * n_units + u

        def idx_copy(s):
            return pltpu.make_async_copy(
                i_hbm.at[0, pl.ds(blk(s) * _W, _W)], idx_v.at[s % 2],
                si[s % 2])

        def out_copy(s):
            return pltpu.make_async_copy(
                scr.at[s % 2], o_hbm.at[pl.ds(blk(s) * _W, _W), :], sw[s % 2])

        @pl.when(blk(0) < n_blocks)
        def _pro():
            idx_copy(0).start()

        for s in range(max_per_unit):
            if s + 1 < max_per_unit:
                @pl.when(blk(s + 1) < n_blocks)
                def _pre(s=s):
                    idx_copy(s + 1).start()

            @pl.when(blk(s) < n_blocks)
            def _main(s=s):
                idx_copy(s).wait()
                if s >= 2:
                    out_copy(s - 2).wait()
                pltpu.async_copy(cb_hbm.at[idx_v.at[s % 2]], scr.at[s % 2],
                                 sg).wait()
                out_copy(s).start()

        for s in range(max(0, max_per_unit - 2), max_per_unit):
            @pl.when(blk(s) < n_blocks)
            def _drain(s=s):
                out_copy(s).wait()

    return _sc_gather(codebook, idx2)[:, :h_out]


def kernel(x, codebook):
    b, m, h = x.shape
    xt = jnp.swapaxes(x, 1, 2)
    cbt = codebook.T
    idx_a, loss_a = _argmin_loss(xt, cbt, 0, 2)
    qt_a = _gather_rows_t(cbt, idx_a, b // 2, m)
    idx_b, loss_b = _argmin_loss(xt, cbt, 1, 2)
    qt = _gather_rows_t(cbt, idx_b, b // 2, m, b0=b // 2, carry=qt_a)
    idx = jnp.concatenate([idx_a, idx_b]).reshape(-1)
    loss = (loss_a[0, 0] + loss_b[0, 0]) * (1.25 / (b * m * h))
    quantized = jnp.swapaxes(qt, 1, 2)
    return quantized, loss, idx


# speedup vs baseline: 1.9673x; 1.9673x over previous
import dataclasses

import jax
import jax.numpy as jnp
from jax.experimental import pallas as pl
from jax.experimental.pallas import tpu as pltpu
from jax.experimental.pallas import tpu_sc as plsc

_SC_PARAMS = pltpu.CompilerParams()
if "needs_layout_passes" in pltpu.CompilerParams.__dataclass_fields__:
    _SC_PARAMS = dataclasses.replace(_SC_PARAMS, needs_layout_passes=False)

_BATCH = 2


def _vq_tc_body(x_ref, cb_ref, idx_ref, loss_ref):
    i = pl.program_id(0)
    xc = jnp.concatenate([x_ref[b] for b in range(_BATCH)], axis=1)
    cb = cb_ref[...]
    n = xc.shape[1]
    mm = jax.lax.dot_general(
        cb, xc, (((0,), (0,)), ((), ())),
        preferred_element_type=jnp.float32)
    xsq = jnp.sum(xc * xc, axis=0, keepdims=True)
    cbsq = jnp.sum(cb * cb, axis=0).reshape(-1, 1)
    s_h = 128
    n_s = mm.shape[0] // s_h

    def slab(s):
        return (xsq - 2.0 * mm[s * s_h:(s + 1) * s_h]) + cbsq[s * s_h:
                                                              (s + 1) * s_h]

    best = slab(0)
    best_s = jnp.zeros(best.shape, jnp.int32)
    for s in range(1, n_s):
        d_s = slab(s)
        upd = d_s < best
        best = jnp.where(upd, d_s, best)
        best_s = jnp.where(upd, jnp.int32(s), best_s)
    m = jnp.min(best, axis=0, keepdims=True)
    sub = jax.lax.broadcasted_iota(jnp.int32, best.shape, 0)
    cand = best_s * s_h + sub
    masked = jnp.where(best == m, cand, jnp.int32(2 ** 30))
    idx_ref[0, 0, :] = jnp.min(masked, axis=0)
    part = jnp.sum(m).reshape(1, 1)

    @pl.when(i == 0)
    def _init():
        loss_ref[...] = part

    @pl.when(i != 0)
    def _acc():
        loss_ref[...] += part

    @pl.when(i == pl.num_programs(0) - 1)
    def _finish():
        total = pl.num_programs(0) * n * x_ref.shape[1]
        loss_ref[...] *= 1.25 / total


def _argmin_loss(xt, cbt, *, interpret=False):
    b, h, m = xt.shape
    grid = b // _BATCH
    return pl.pallas_call(
        _vq_tc_body,
        grid=(grid,),
        in_specs=[
            pl.BlockSpec((_BATCH, h, m), lambda i: (i, 0, 0)),
            pl.BlockSpec(cbt.shape, lambda i: (0, 0)),
        ],
        out_specs=[
            pl.BlockSpec((1, 1, _BATCH * m), lambda i: (i, 0, 0)),
            pl.BlockSpec((1, 1), lambda i: (0, 0)),
        ],
        out_shape=[
            jax.ShapeDtypeStruct((grid, 1, _BATCH * m), jnp.int32),
            jax.ShapeDtypeStruct((1, 1), jnp.float32),
        ],
        interpret=interpret,
    )(xt, cbt)


def _gather_rows_t(cbt, idx2, b, m):
    h = cbt.shape[0]
    n_bp = idx2.shape[0]
    w = idx2.shape[2]
    mesh = plsc.VectorSubcoreMesh(core_axis_name="core",
                                  subcore_axis_name="subcore")
    n_hg = h // 8
    bp_per_u = n_bp // 4

    @pl.kernel(out_type=jax.ShapeDtypeStruct((b, h, m), cbt.dtype),
               mesh=mesh,
               compiler_params=_SC_PARAMS,
               scratch_types=[pltpu.VMEM((8, cbt.shape[1]), cbt.dtype),
                              pltpu.VMEM((1, 1, w), jnp.int32),
                              pltpu.VMEM((1, 8, m), cbt.dtype),
                              pltpu.VMEM((1, 8, m), cbt.dtype),
                              pltpu.SemaphoreType.DMA,
                              pltpu.SemaphoreType.DMA,
                              pltpu.SemaphoreType.DMA,
                              pltpu.SemaphoreType.DMA])
    def _sc_gather(cb_hbm, i_hbm, o_hbm, cb_v, idx_v, ob0, ob1, sc, si,
                   sw0, sw1):
        u = (jax.lax.axis_index("core") * 16
             + jax.lax.axis_index("subcore"))
        hg = u % n_hg
        bpg = u // n_hg
        obufs = (ob0, ob1)
        sw = (sw0, sw1)
        pltpu.async_copy(cb_hbm.at[pl.ds(hg * 8, 8), :], cb_v, sc).wait()

        def out_copy(bp, bi):
            return pltpu.make_async_copy(
                obufs[bi],
                o_hbm.at[pl.ds(2 * bp + bi, 1), pl.ds(hg * 8, 8), :],
                sw[bi])

        @pl.loop(0, bp_per_u)
        def _chunk(t):
            bp = bpg * bp_per_u + t
            pltpu.async_copy(i_hbm.at[pl.ds(bp, 1), :, :], idx_v, si).wait()
            for bi in range(2):
                @pl.when(t >= 1)
                def _wait_prev(bi=bi):
                    out_copy(bp - 1, bi).wait()
                for hh in range(8):
                    hh_vec = jnp.full((16,), hh, jnp.int32)
                    for j in range(m // 16):
                        g = plsc.load_gather(
                            cb_v,
                            [hh_vec,
                             idx_v[0, 0, pl.ds(bi * m + j * 16, 16)]])
                        obufs[bi][0, hh, pl.ds(j * 16, 16)] = g
                out_copy(bp, bi).start()

        for bi in range(2):
            out_copy(bpg * bp_per_u + bp_per_u - 1, bi).wait()

    return _sc_gather(cbt, idx2)


def kernel(x, codebook):
    b, m, h = x.shape
    xt = jnp.swapaxes(x, 1, 2)
    cbt = codebook.T
    idx3, loss = _argmin_loss(xt, cbt)
    idx = idx3.reshape(-1)
    qt = _gather_rows_t(cbt, idx3, b, m)
    quantized = jnp.swapaxes(qt, 1, 2)
    return quantized, loss[0, 0], idx
